# baseline (speedup 1.0000x reference)
import functools
import math

import jax
import jax.numpy as jnp
import numpy as np
from jax import lax
from jax.experimental import pallas as pl
from jax.experimental.pallas import tpu as pltpu
from jax.experimental.pallas import tpu_sc as plsc

_D_MODEL = 1024
_SEQ_LEN = 2048
_BATCH = 4
_MAX_LEN = 5000

_NC = 2
_NS = 16
_NW = _NC * _NS

_POS_PER_W = _SEQ_LEN // _NW
_CHUNK = 8
_NCHUNK = _POS_PER_W // _CHUNK
_KUNROLL = 8


def _make_pe(d_model, seq_len):
    position = np.arange(_MAX_LEN)[:, np.newaxis]
    div_term = np.exp(np.arange(0, d_model, 2) * (-math.log(10000.0) / d_model))
    pe = np.zeros((_MAX_LEN, d_model))
    pe[:, 0::2] = np.sin(position * div_term)
    pe[:, 1::2] = np.cos(position * div_term)
    return pe[:seq_len].astype(np.float32)


_PE = _make_pe(_D_MODEL, _SEQ_LEN)

_ROWS_VMEM = [pltpu.VMEM((_CHUNK, _D_MODEL), jnp.float32)
              for _ in range(2 * _BATCH)]
_PE_VMEM = [pltpu.VMEM((_CHUNK, _D_MODEL), jnp.float32) for _ in range(2)]
_SEMS = [pltpu.SemaphoreType.DMA for _ in range(2 * _BATCH * 2 + 2)]


@functools.partial(
    pl.kernel,
    mesh=plsc.VectorSubcoreMesh(core_axis_name="c", subcore_axis_name="s"),
    out_type=jax.ShapeDtypeStruct((_BATCH * _SEQ_LEN, _D_MODEL), jnp.float32),
    scratch_types=[pltpu.VMEM((_NCHUNK, _BATCH, _CHUNK), jnp.int32)]
    + _ROWS_VMEM + _PE_VMEM + _SEMS,
)
def _sc_embed(idx_hbm, table_hbm, pe_hbm, out_hbm, idx_v,
              ra0, ra1, ra2, ra3, rb0, rb1, rb2, rb3, pa, pb,
              gsa0, gsa1, gsa2, gsa3, gsb0, gsb1, gsb2, gsb3,
              osa0, osa1, osa2, osa3, osb0, osb1, osb2, osb3,
              psa, psb):
    wid = lax.axis_index("s") * _NC + lax.axis_index("c")
    pos0 = wid * _POS_PER_W

    rows = ((ra0, ra1, ra2, ra3), (rb0, rb1, rb2, rb3))
    pebuf = (pa, pb)
    gsem = ((gsa0, gsa1, gsa2, gsa3), (gsb0, gsb1, gsb2, gsb3))
    osem = ((osa0, osa1, osa2, osa3), (osb0, osb1, osb2, osb3))
    psem = (psa, psb)

    pltpu.sync_copy(idx_hbm.at[wid], idx_v)

    def issue_chunk(q, s):
        pe_cp = pltpu.async_copy(
            pe_hbm.at[pl.ds(pos0 + q * _CHUNK, _CHUNK)], pebuf[s], psem[s])
        g_cps = [
            pltpu.async_copy(table_hbm.at[idx_v.at[q, b]], rows[s][b],
                             gsem[s][b])
            for b in range(_BATCH)
        ]
        return pe_cp, g_cps

    inflight = [None, None]
    outflight = [None, None]
    inflight[0] = issue_chunk(0, 0)

    for q in range(_NCHUNK):
        s = q % 2
        o = 1 - s
        pe_cp, g_cps = inflight[s]
        pe_cp.wait()
        for cp in g_cps:
            cp.wait()

        if q + 1 < _NCHUNK:
            if outflight[o] is not None:
                for cp in outflight[o]:
                    cp.wait()
                outflight[o] = None
            inflight[o] = issue_chunk(q + 1, o)

        r0, r1, r2, r3 = rows[s]
        pv = pebuf[s]

        def _add_row(r, _):
            def _add_k(k8, _):
                for u in range(_KUNROLL):
                    sl = pl.ds(k8 * (16 * _KUNROLL) + u * 16, 16)
                    v = pv[r, sl]
                    r0[r, sl] = r0[r, sl] + v
                    r1[r, sl] = r1[r, sl] + v
                    r2[r, sl] = r2[r, sl] + v
                    r3[r, sl] = r3[r, sl] + v
                return 0

            lax.fori_loop(0, _D_MODEL // (16 * _KUNROLL), _add_k, 0)
            return 0

        lax.fori_loop(0, _CHUNK, _add_row, 0)

        outflight[s] = [
            pltpu.async_copy(
                rows[s][b],
                out_hbm.at[pl.ds(b * _SEQ_LEN + pos0 + q * _CHUNK, _CHUNK)],
                osem[s][b])
            for b in range(_BATCH)
        ]

    for fl in outflight:
        if fl is not None:
            for cp in fl:
                cp.wait()


def kernel(x, emb_table):
    batch, seq_len = x.shape
    d_model = emb_table.shape[1]
    idx = (x.astype(jnp.int32)
           .reshape(batch, _NW, _NCHUNK, _CHUNK)
           .transpose(1, 2, 0, 3))
    out = _sc_embed(idx, emb_table, jnp.asarray(_PE))
    return out.reshape(batch, seq_len, d_model)


# speedup vs baseline: 1.8302x; 1.8302x over previous
import functools
import math

import jax
import jax.numpy as jnp
import numpy as np
from jax import lax
from jax.experimental import pallas as pl
from jax.experimental.pallas import tpu as pltpu
from jax.experimental.pallas import tpu_sc as plsc

_D_MODEL = 1024
_SEQ_LEN = 2048
_BATCH = 4
_MAX_LEN = 5000

_NC = 2
_NS = 16
_NW = _NC * _NS

_POS_PER_W = _SEQ_LEN // _NW
_CHUNK = 16
_NCHUNK = _POS_PER_W // _CHUNK
_NSTEP = _NCHUNK * _BATCH
_VECS_PER_ROW = _D_MODEL // 16


def _make_pe(d_model, seq_len):
    position = np.arange(_MAX_LEN)[:, np.newaxis]
    div_term = np.exp(np.arange(0, d_model, 2) * (-math.log(10000.0) / d_model))
    pe = np.zeros((_MAX_LEN, d_model))
    pe[:, 0::2] = np.sin(position * div_term)
    pe[:, 1::2] = np.cos(position * div_term)
    return pe[:seq_len].astype(np.float32)


_PE = _make_pe(_D_MODEL, _SEQ_LEN)


def _pack_pe_words(pe):
    import ml_dtypes
    seq_len, d_model = pe.shape
    bits = pe.astype(ml_dtypes.bfloat16).view(np.uint16).astype(np.uint32)
    g = bits.reshape(seq_len, d_model // 32, 2, 16)
    words = g[:, :, 0, :] | (g[:, :, 1, :] << 16)
    return np.ascontiguousarray(
        words.reshape(seq_len, d_model // 2)).view(np.int32)


_PE_W = _pack_pe_words(_PE)


@functools.partial(
    pl.kernel,
    mesh=plsc.VectorSubcoreMesh(core_axis_name="c", subcore_axis_name="s"),
    out_type=jax.ShapeDtypeStruct((_BATCH * _SEQ_LEN, _D_MODEL), jnp.float32),
    scratch_types=[
        pltpu.VMEM((_NSTEP, _CHUNK), jnp.int32),
        pltpu.VMEM((_CHUNK, _D_MODEL), jnp.float32),
        pltpu.VMEM((_CHUNK, _D_MODEL), jnp.float32),
        pltpu.VMEM((_CHUNK, _D_MODEL), jnp.float32),
        pltpu.VMEM((_CHUNK, _D_MODEL), jnp.float32),
        pltpu.VMEM((_CHUNK, _D_MODEL), jnp.float32),
        pltpu.VMEM((_CHUNK, _D_MODEL), jnp.float32),
        pltpu.SemaphoreType.DMA,
        pltpu.SemaphoreType.DMA,
        pltpu.SemaphoreType.DMA,
        pltpu.SemaphoreType.DMA,
        pltpu.SemaphoreType.DMA,
        pltpu.SemaphoreType.DMA,
        pltpu.SemaphoreType.DMA,
        pltpu.SemaphoreType.DMA,
        pltpu.SemaphoreType.DMA,
        pltpu.SemaphoreType.DMA,
    ],
)
def _sc_embed(idx_hbm, table_hbm, pe_hbm, out_hbm,
              idx_v, r0, r1, r2, r3, p0, p1,
              gs0, gs1, gs2, gs3, os0, os1, os2, os3, ps0, ps1):
    wid = lax.axis_index("s") * _NC + lax.axis_index("c")
    pos0 = wid * _POS_PER_W

    rbuf = (r0, r1, r2, r3)
    pbuf = (p0, p1)
    gsem = (gs0, gs1, gs2, gs3)
    osem = (os0, os1, os2, os3)
    psem = (ps0, ps1)
    nbuf = 4

    pltpu.sync_copy(idx_hbm.at[wid], idx_v)

    pe_cp = [None, None]
    pe_cp[0] = pltpu.async_copy(pe_hbm.at[pl.ds(pos0, _CHUNK)], p0, ps0)
    g_cp = [None] * _NSTEP
    o_cp = [None] * _NSTEP
    lead = 2
    for t in range(lead):
        g_cp[t] = pltpu.async_copy(
            table_hbm.at[idx_v.at[t]], rbuf[t], gsem[t])

    for s in range(_NSTEP):
        c, b = divmod(s, _BATCH)
        g_cp[s].wait()
        if b == 0:
            pe_cp[c % 2].wait()

        rb = rbuf[s % nbuf]
        pb = pbuf[c % 2]

        def _add_row(r, _):
            for k in range(_VECS_PER_ROW):
                sl = pl.ds(k * 16, 16)
                rb[r, sl] = rb[r, sl] + pb[r, sl]
            return 0

        lax.fori_loop(0, _CHUNK, _add_row, 0)
        o_cp[s] = pltpu.async_copy(
            rb, out_hbm.at[pl.ds(b * _SEQ_LEN + pos0 + c * _CHUNK, _CHUNK)],
            osem[s % nbuf])

        t = s + lead
        if t < _NSTEP:
            if t - nbuf >= 0:
                o_cp[t - nbuf].wait()
            c1, b1 = divmod(t, _BATCH)
            if b1 == 0:
                pe_cp[c1 % 2] = pltpu.async_copy(
                    pe_hbm.at[pl.ds(pos0 + c1 * _CHUNK, _CHUNK)],
                    pbuf[c1 % 2], psem[c1 % 2])
            g_cp[t] = pltpu.async_copy(
                table_hbm.at[idx_v.at[t]], rbuf[t % nbuf], gsem[t % nbuf])

    for s in range(_NSTEP - nbuf, _NSTEP):
        o_cp[s].wait()


def kernel(x, emb_table):
    batch, seq_len = x.shape
    d_model = emb_table.shape[1]
    idx = (x.astype(jnp.int32)
           .reshape(batch, _NW, _NCHUNK, _CHUNK)
           .transpose(1, 2, 0, 3)
           .reshape(_NW, _NSTEP, _CHUNK))
    out = _sc_embed(idx, emb_table, jnp.asarray(_PE))
    return out.reshape(batch, seq_len, d_model)


# speedup vs baseline: 1.8632x; 1.0180x over previous
import functools
import math

import jax
import jax.numpy as jnp
import numpy as np
from jax import lax
from jax.experimental import pallas as pl
from jax.experimental.pallas import tpu as pltpu
from jax.experimental.pallas import tpu_sc as plsc

_D_MODEL = 1024
_SEQ_LEN = 2048
_BATCH = 4
_MAX_LEN = 5000

_NC = 2
_NS = 16
_NW = _NC * _NS

_POS_PER_W = _SEQ_LEN // _NW
_CHUNK = 16
_NCHUNK = _POS_PER_W // _CHUNK
_NSTEP = _NCHUNK * _BATCH
_VECS_PER_ROW = _D_MODEL // 16


def _make_pe(d_model, seq_len):
    position = np.arange(_MAX_LEN)[:, np.newaxis]
    div_term = np.exp(np.arange(0, d_model, 2) * (-math.log(10000.0) / d_model))
    pe = np.zeros((_MAX_LEN, d_model))
    pe[:, 0::2] = np.sin(position * div_term)
    pe[:, 1::2] = np.cos(position * div_term)
    return pe[:seq_len].astype(np.float32)


_PE = _make_pe(_D_MODEL, _SEQ_LEN)


def _pack_pe_words(pe):
    import ml_dtypes
    seq_len, d_model = pe.shape
    bits = pe.astype(ml_dtypes.bfloat16).view(np.uint16).astype(np.uint32)
    g = bits.reshape(seq_len, d_model // 32, 2, 16)
    words = g[:, :, 0, :] | (g[:, :, 1, :] << 16)
    return np.ascontiguousarray(
        words.reshape(seq_len, d_model // 2)).view(np.int32)


_PE_W = _pack_pe_words(_PE)


@functools.partial(
    pl.kernel,
    mesh=plsc.VectorSubcoreMesh(core_axis_name="c", subcore_axis_name="s"),
    out_type=jax.ShapeDtypeStruct((_BATCH * _SEQ_LEN, _D_MODEL), jnp.float32),
    scratch_types=[
        pltpu.VMEM((_NSTEP, _CHUNK), jnp.int32),
        pltpu.VMEM((_CHUNK, _D_MODEL), jnp.float32),
        pltpu.VMEM((_CHUNK, _D_MODEL), jnp.float32),
        pltpu.VMEM((_CHUNK, _D_MODEL), jnp.float32),
        pltpu.VMEM((_CHUNK, _D_MODEL), jnp.float32),
        pltpu.VMEM((_CHUNK, _D_MODEL), jnp.float32),
        pltpu.VMEM((_CHUNK, _D_MODEL), jnp.float32),
        pltpu.VMEM((_CHUNK, _D_MODEL), jnp.float32),
        pltpu.SemaphoreType.DMA,
        pltpu.SemaphoreType.DMA,
        pltpu.SemaphoreType.DMA,
        pltpu.SemaphoreType.DMA,
        pltpu.SemaphoreType.DMA,
        pltpu.SemaphoreType.DMA,
        pltpu.SemaphoreType.DMA,
        pltpu.SemaphoreType.DMA,
        pltpu.SemaphoreType.DMA,
        pltpu.SemaphoreType.DMA,
        pltpu.SemaphoreType.DMA,
        pltpu.SemaphoreType.DMA,
    ],
)
def _sc_embed(idx_hbm, table_hbm, pe_hbm, out_hbm,
              idx_v, r0, r1, r2, r3, r4, p0, p1,
              gs0, gs1, gs2, gs3, gs4, os0, os1, os2, os3, os4, ps0, ps1):
    wid = lax.axis_index("s") * _NC + lax.axis_index("c")
    pos0 = wid * _POS_PER_W

    rbuf = (r0, r1, r2, r3, r4)
    pbuf = (p0, p1)
    gsem = (gs0, gs1, gs2, gs3, gs4)
    osem = (os0, os1, os2, os3, os4)
    psem = (ps0, ps1)
    nbuf = 5

    pltpu.sync_copy(idx_hbm.at[wid], idx_v)

    pe_cp = [None, None]
    pe_cp[0] = pltpu.async_copy(pe_hbm.at[pl.ds(pos0, _CHUNK)], p0, ps0)
    g_cp = [None] * _NSTEP
    o_cp = [None] * _NSTEP
    lead = 4
    for t in range(lead):
        g_cp[t] = pltpu.async_copy(
            table_hbm.at[idx_v.at[t]], rbuf[t], gsem[t])

    for s in range(_NSTEP):
        c, b = divmod(s, _BATCH)
        g_cp[s].wait()
        if b == 0:
            pe_cp[c % 2].wait()

        rb = rbuf[s % nbuf]
        pb = pbuf[c % 2]

        def _add_row(r, _):
            for k in range(_VECS_PER_ROW):
                sl = pl.ds(k * 16, 16)
                rb[r, sl] = rb[r, sl] + pb[r, sl]
            return 0

        lax.fori_loop(0, _CHUNK, _add_row, 0)
        o_cp[s] = pltpu.async_copy(
            rb, out_hbm.at[pl.ds(b * _SEQ_LEN + pos0 + c * _CHUNK, _CHUNK)],
            osem[s % nbuf])

        t = s + lead
        if t < _NSTEP:
            if t - nbuf >= 0:
                o_cp[t - nbuf].wait()
            c1, b1 = divmod(t, _BATCH)
            if b1 == 0:
                pe_cp[c1 % 2] = pltpu.async_copy(
                    pe_hbm.at[pl.ds(pos0 + c1 * _CHUNK, _CHUNK)],
                    pbuf[c1 % 2], psem[c1 % 2])
            g_cp[t] = pltpu.async_copy(
                table_hbm.at[idx_v.at[t]], rbuf[t % nbuf], gsem[t % nbuf])

    for s in range(_NSTEP - nbuf, _NSTEP):
        o_cp[s].wait()


def kernel(x, emb_table):
    batch, seq_len = x.shape
    d_model = emb_table.shape[1]
    idx = (x.astype(jnp.int32)
           .reshape(batch, _NW, _NCHUNK, _CHUNK)
           .transpose(1, 2, 0, 3)
           .reshape(_NW, _NSTEP, _CHUNK))
    out = _sc_embed(idx, emb_table, jnp.asarray(_PE))
    return out.reshape(batch, seq_len, d_model)
